# Initial kernel scaffold; baseline (speedup 1.0000x reference)
import jax
import jax.numpy as jnp
from jax.experimental import pallas as pl


def kernel(Z, embeddings):
    raise NotImplementedError("write your pallas kernel here")


# speedup vs baseline: 1.1028x; 1.1028x over previous
import functools

import jax
import jax.numpy as jnp
from jax import lax
from jax.experimental import pallas as pl
from jax.experimental.pallas import tpu as pltpu
from jax.experimental.pallas import tpu_sc as plsc

D = 32
NC = 2
NS = 16
NW = NC * NS
CHUNK = 1600


def _make_gather(B):
    b_per_w = B // NW
    nchunk = b_per_w // CHUNK
    mesh = plsc.VectorSubcoreMesh(core_axis_name="c", subcore_axis_name="s")

    @functools.partial(
        pl.kernel,
        mesh=mesh,
        out_type=jax.ShapeDtypeStruct((B, D), jnp.float32),
        compiler_params=pltpu.CompilerParams(use_tc_tiling_on_sc=False),
        scratch_types=[
            pltpu.VMEM((CHUNK,), jnp.int32),
            pltpu.VMEM((CHUNK, D), jnp.float32),
            pltpu.SemaphoreType.DMA,
        ],
    )
    def k(table_hbm, idx_hbm, out_hbm, idx_v, rows_v, sem):
        wid = lax.axis_index("s") * NC + lax.axis_index("c")
        base = wid * b_per_w

        def body(g, carry):
            off = base + g * CHUNK
            pltpu.sync_copy(idx_hbm.at[pl.ds(off, CHUNK)], idx_v)
            pltpu.async_copy(table_hbm.at[idx_v], rows_v, sem).wait()
            pltpu.sync_copy(rows_v, out_hbm.at[pl.ds(off, CHUNK)])
            return carry

        lax.fori_loop(0, nchunk, body, 0)

    return k


def kernel(Z, embeddings):
    B = Z.shape[0] * Z.shape[1]
    flat = Z.reshape(B)
    out = _make_gather(B)(embeddings, flat)
    return out.reshape(Z.shape[0], Z.shape[1], D)


# speedup vs baseline: 1.1101x; 1.0067x over previous
import functools

import jax
import jax.numpy as jnp
from jax import lax
from jax.experimental import pallas as pl
from jax.experimental.pallas import tpu as pltpu
from jax.experimental.pallas import tpu_sc as plsc

D = 32
NC = 2
NS = 16
NW = NC * NS
CHUNK = 1600


def _make_gather(B):
    b_per_w = B // NW
    nchunk = b_per_w // CHUNK
    assert nchunk % 2 == 0 and nchunk >= 4
    mesh = plsc.VectorSubcoreMesh(core_axis_name="c", subcore_axis_name="s")

    @functools.partial(
        pl.kernel,
        mesh=mesh,
        out_type=jax.ShapeDtypeStruct((B, D), jnp.float32),
        compiler_params=pltpu.CompilerParams(use_tc_tiling_on_sc=False),
        scratch_types=[
            pltpu.VMEM((b_per_w,), jnp.int32),
            pltpu.VMEM((CHUNK, D), jnp.float32),
            pltpu.VMEM((CHUNK, D), jnp.float32),
            pltpu.SemaphoreType.DMA,
            pltpu.SemaphoreType.DMA,
            pltpu.SemaphoreType.DMA,
            pltpu.SemaphoreType.DMA,
        ],
    )
    def k(table, idxh, outh, idx_v, rows0, rows1, g0s, g1s, s0s, s1s):
        wid = lax.axis_index("s") * NC + lax.axis_index("c")
        base = wid * b_per_w
        pltpu.sync_copy(idxh.at[pl.ds(base, b_per_w)], idx_v)

        def gcopy(g, rows, sem):
            return pltpu.make_async_copy(
                table.at[idx_v.at[pl.ds(g * CHUNK, CHUNK)]], rows, sem)

        def scopy(g, rows, sem):
            return pltpu.make_async_copy(
                rows, outh.at[pl.ds(base + g * CHUNK, CHUNK)], sem)

        gcopy(0, rows0, g0s).start()
        gcopy(1, rows1, g1s).start()

        def body(t, carry):
            g0 = 2 * t
            g1 = g0 + 1
            gcopy(g0, rows0, g0s).wait()
            scopy(g0, rows0, s0s).start()
            gcopy(g1, rows1, g1s).wait()
            scopy(g1, rows1, s1s).start()
            scopy(g0, rows0, s0s).wait()
            gcopy(g0 + 2, rows0, g0s).start()
            scopy(g1, rows1, s1s).wait()
            gcopy(g1 + 2, rows1, g1s).start()
            return carry

        lax.fori_loop(0, nchunk // 2 - 1, body, 0)

        gl0 = nchunk - 2
        gl1 = nchunk - 1
        gcopy(gl0, rows0, g0s).wait()
        scopy(gl0, rows0, s0s).start()
        gcopy(gl1, rows1, g1s).wait()
        scopy(gl1, rows1, s1s).start()
        scopy(gl0, rows0, s0s).wait()
        scopy(gl1, rows1, s1s).wait()

    return k


def kernel(Z, embeddings):
    B = Z.shape[0] * Z.shape[1]
    flat = Z.reshape(B)
    out = _make_gather(B)(embeddings, flat)
    return out.reshape(Z.shape[0], Z.shape[1], D)


# speedup vs baseline: 1.8047x; 1.6256x over previous
import functools

import jax
import jax.numpy as jnp
from jax import lax
from jax.experimental import pallas as pl
from jax.experimental.pallas import tpu as pltpu
from jax.experimental.pallas import tpu_sc as plsc

D = 32
NC = 2
NS = 16
NW = NC * NS
G = 50
CI = 32
CHUNK = CI * G


def _make_gather(NI):
    b_per_w = NI * G // NW
    i_per_w = NI // NW
    nchunk = i_per_w // CI
    assert nchunk % 2 == 0 and nchunk >= 4
    mesh = plsc.VectorSubcoreMesh(core_axis_name="c", subcore_axis_name="s")

    @functools.partial(
        pl.kernel,
        mesh=mesh,
        out_type=jax.ShapeDtypeStruct((NI, G, D), jnp.float32),
        compiler_params=pltpu.CompilerParams(use_tc_tiling_on_sc=False),
        scratch_types=[
            pltpu.VMEM((b_per_w,), jnp.int32),
            pltpu.VMEM((CHUNK, D), jnp.float32),
            pltpu.VMEM((CHUNK, D), jnp.float32),
            pltpu.SemaphoreType.DMA,
            pltpu.SemaphoreType.DMA,
            pltpu.SemaphoreType.DMA,
            pltpu.SemaphoreType.DMA,
        ],
    )
    def k(table, idxh, outh, idx_v, rows0, rows1, g0s, g1s, s0s, s1s):
        wid = lax.axis_index("s") * NC + lax.axis_index("c")
        base = wid * b_per_w
        ibase = wid * i_per_w
        pltpu.sync_copy(idxh.at[pl.ds(base, b_per_w)], idx_v)

        def gcopy(g, rows, sem):
            return pltpu.make_async_copy(
                table.at[idx_v.at[pl.ds(g * CHUNK, CHUNK)]], rows, sem)

        def store_start(g, rows, sem):
            i0 = ibase + g * CI
            for ki in range(CI):
                pltpu.make_async_copy(
                    rows.at[pl.ds(ki * G, G)], outh.at[i0 + ki], sem).start()

        def store_wait(g, rows, sem):
            i0 = ibase + g * CI
            for ki in range(CI):
                pltpu.make_async_copy(
                    rows.at[pl.ds(ki * G, G)], outh.at[i0 + ki], sem).wait()

        gcopy(0, rows0, g0s).start()
        gcopy(1, rows1, g1s).start()

        def body(t, carry):
            g0 = 2 * t
            g1 = g0 + 1
            gcopy(g0, rows0, g0s).wait()
            store_start(g0, rows0, s0s)
            gcopy(g1, rows1, g1s).wait()
            store_start(g1, rows1, s1s)
            store_wait(g0, rows0, s0s)
            gcopy(g0 + 2, rows0, g0s).start()
            store_wait(g1, rows1, s1s)
            gcopy(g1 + 2, rows1, g1s).start()
            return carry

        lax.fori_loop(0, nchunk // 2 - 1, body, 0)

        gl0 = nchunk - 2
        gl1 = nchunk - 1
        gcopy(gl0, rows0, g0s).wait()
        store_start(gl0, rows0, s0s)
        gcopy(gl1, rows1, g1s).wait()
        store_start(gl1, rows1, s1s)
        store_wait(gl0, rows0, s0s)
        store_wait(gl1, rows1, s1s)

    return k


def kernel(Z, embeddings):
    NI = Z.shape[0]
    flat = Z.reshape(NI * G)
    return _make_gather(NI)(embeddings, flat)
